# baseline (speedup 1.0000x reference)
import jax
import jax.numpy as jnp
from jax.experimental import pallas as pl
from jax.experimental.pallas import tpu as pltpu

_BM = 200
_CACHE_BLOCKS = 9


def _s1_kernel(x_ref, w1_ref, o_ref):
    o_ref[...] = jnp.dot(x_ref[...], w1_ref[...],
                         preferred_element_type=jnp.float32
                         ).astype(jnp.bfloat16)


def _fused_kernel(adj_hbm, s1_ref, b1_ref, w2_ref, b2_ref,
                  o_ref, t2_ref, cache_ref, buf_ref, sem_ref):
    p = pl.program_id(0)
    i = pl.program_id(1)
    nb = pl.num_programs(1)
    bm = o_ref.shape[0]
    c2 = 2 * _CACHE_BLOCKS
    s = p * nb + i
    cur_slot = jax.lax.rem(s, 2)
    nxt_slot = jax.lax.rem(s + 1, 2)

    def block_dma(b, slot):
        return pltpu.make_async_copy(
            adj_hbm.at[pl.ds(b * bm, bm), :],
            buf_ref.at[slot],
            sem_ref.at[slot])

    @pl.when(s == 0)
    def _prime():
        block_dma(0, 0).start()

    ni = jnp.where(i == nb - 1, 0, i + 1)
    nph = jnp.where(i == nb - 1, p + 1, p)
    nxt_cached = (nph == 1) & (ni % 2 == 1) & (ni < c2)
    nxt_valid = jnp.logical_not((p == 1) & (i == nb - 1))

    @pl.when(nxt_valid & jnp.logical_not(nxt_cached))
    def _prefetch():
        block_dma(ni, nxt_slot).start()

    is_cached = (i % 2 == 1) & (i < c2)

    @pl.when(p == 0)
    def _layer1():
        block_dma(i, cur_slot).wait()

        @pl.when(is_cached)
        def _stash():
            cache_ref[i // 2] = buf_ref[cur_slot].astype(jnp.bfloat16)

        acc = jnp.dot(buf_ref[cur_slot].astype(jnp.bfloat16), s1_ref[...],
                      preferred_element_type=jnp.float32)
        h = jnp.maximum(acc + b1_ref[...], 0.0)
        t2_ref[pl.ds(i * bm, bm), :] = jnp.dot(
            h, w2_ref[...], preferred_element_type=jnp.float32)

    def _layer2_epilogue(a):
        z = jnp.dot(a, t2_ref[...].astype(jnp.bfloat16),
                    preferred_element_type=jnp.float32) + b2_ref[...]
        m = jnp.max(z, axis=1, keepdims=True)
        e = jnp.exp(z - m)
        lse = jnp.log(jnp.sum(e, axis=1, keepdims=True)) + m
        o_ref[...] = z - lse

    @pl.when((p == 1) & is_cached)
    def _layer2_cached():
        _layer2_epilogue(cache_ref[i // 2])

    @pl.when((p == 1) & jnp.logical_not(is_cached))
    def _layer2_streamed():
        block_dma(i, cur_slot).wait()
        _layer2_epilogue(buf_ref[cur_slot].astype(jnp.bfloat16))


def kernel(x, adj, W1, b1, W2, b2):
    n, nfeat = x.shape
    nhid = W1.shape[1]
    ncls = W2.shape[1]
    bm = _BM
    b1r = b1.reshape(1, nhid)
    b2r = b2.reshape(1, ncls)
    BS1 = 2000

    s1 = pl.pallas_call(
        _s1_kernel,
        grid=(n // BS1,),
        in_specs=[pl.BlockSpec((BS1, nfeat), lambda i: (i, 0)),
                  pl.BlockSpec((nfeat, nhid), lambda i: (0, 0))],
        out_specs=pl.BlockSpec((BS1, nhid), lambda i: (i, 0)),
        out_shape=jax.ShapeDtypeStruct((n, nhid), jnp.bfloat16),
    )(x, W1)

    return pl.pallas_call(
        _fused_kernel,
        grid=(2, n // bm),
        in_specs=[pl.BlockSpec(memory_space=pltpu.MemorySpace.HBM),
                  pl.BlockSpec((n, nhid), lambda p, i: (0, 0)),
                  pl.BlockSpec((1, nhid), lambda p, i: (0, 0)),
                  pl.BlockSpec((nhid, ncls), lambda p, i: (0, 0)),
                  pl.BlockSpec((1, ncls), lambda p, i: (0, 0))],
        out_specs=pl.BlockSpec((bm, ncls), lambda p, i: (i, 0)),
        out_shape=jax.ShapeDtypeStruct((n, ncls), jnp.float32),
        scratch_shapes=[pltpu.VMEM((n, ncls), jnp.float32),
                        pltpu.VMEM((_CACHE_BLOCKS, bm, n), jnp.bfloat16),
                        pltpu.VMEM((2, bm, n), jnp.float32),
                        pltpu.SemaphoreType.DMA((2,))],
        compiler_params=pltpu.CompilerParams(
            dimension_semantics=("arbitrary", "arbitrary"),
            vmem_limit_bytes=64 * 1024 * 1024),
    )(adj, s1, b1r, W2, b2r)


# speedup vs baseline: 1.0019x; 1.0019x over previous
import jax
import jax.numpy as jnp
from jax.experimental import pallas as pl
from jax.experimental.pallas import tpu as pltpu

_BM = 200
_CACHE_BLOCKS = 9


def _s1_kernel(x_ref, w1_ref, o_ref):
    o_ref[...] = jnp.dot(x_ref[...], w1_ref[...],
                         preferred_element_type=jnp.float32)


def _fused_kernel(adj_hbm, s1_ref, b1_ref, w2_ref, b2_ref,
                  o_ref, t2_ref, cache_ref, buf_ref, sem_ref):
    p = pl.program_id(0)
    i = pl.program_id(1)
    nb = pl.num_programs(1)
    bm = o_ref.shape[0]
    c2 = 2 * _CACHE_BLOCKS
    s = p * nb + i
    cur_slot = jax.lax.rem(s, 2)
    nxt_slot = jax.lax.rem(s + 1, 2)

    def block_dma(b, slot):
        return pltpu.make_async_copy(
            adj_hbm.at[pl.ds(b * bm, bm), :],
            buf_ref.at[slot],
            sem_ref.at[slot])

    @pl.when(s == 0)
    def _prime():
        block_dma(0, 0).start()

    ni = jnp.where(i == nb - 1, 0, i + 1)
    nph = jnp.where(i == nb - 1, p + 1, p)
    nxt_cached = (nph == 1) & (ni % 2 == 1) & (ni < c2)
    nxt_valid = jnp.logical_not((p == 1) & (i == nb - 1))

    @pl.when(nxt_valid & jnp.logical_not(nxt_cached))
    def _prefetch():
        block_dma(ni, nxt_slot).start()

    is_cached = (i % 2 == 1) & (i < c2)

    @pl.when(p == 0)
    def _layer1():
        block_dma(i, cur_slot).wait()

        @pl.when(is_cached)
        def _stash():
            cache_ref[i // 2] = buf_ref[cur_slot].astype(jnp.bfloat16)

        acc = jnp.dot(buf_ref[cur_slot], s1_ref[...],
                      preferred_element_type=jnp.float32)
        h = jnp.maximum(acc + b1_ref[...], 0.0)
        t2_ref[pl.ds(i * bm, bm), :] = jnp.dot(
            h, w2_ref[...], preferred_element_type=jnp.float32)

    def _layer2_epilogue(zz):
        z = zz + b2_ref[...]
        m = jnp.max(z, axis=1, keepdims=True)
        e = jnp.exp(z - m)
        lse = jnp.log(jnp.sum(e, axis=1, keepdims=True)) + m
        o_ref[...] = z - lse

    @pl.when((p == 1) & is_cached)
    def _layer2_cached():
        _layer2_epilogue(jnp.dot(cache_ref[i // 2],
                                 t2_ref[...].astype(jnp.bfloat16),
                                 preferred_element_type=jnp.float32))

    @pl.when((p == 1) & jnp.logical_not(is_cached))
    def _layer2_streamed():
        block_dma(i, cur_slot).wait()
        _layer2_epilogue(jnp.dot(buf_ref[cur_slot], t2_ref[...],
                                 preferred_element_type=jnp.float32))


def kernel(x, adj, W1, b1, W2, b2):
    n, nfeat = x.shape
    nhid = W1.shape[1]
    ncls = W2.shape[1]
    bm = _BM
    b1r = b1.reshape(1, nhid)
    b2r = b2.reshape(1, ncls)
    BS1 = 2000

    s1 = pl.pallas_call(
        _s1_kernel,
        grid=(n // BS1,),
        in_specs=[pl.BlockSpec((BS1, nfeat), lambda i: (i, 0)),
                  pl.BlockSpec((nfeat, nhid), lambda i: (0, 0))],
        out_specs=pl.BlockSpec((BS1, nhid), lambda i: (i, 0)),
        out_shape=jax.ShapeDtypeStruct((n, nhid), jnp.float32),
    )(x, W1)

    return pl.pallas_call(
        _fused_kernel,
        grid=(2, n // bm),
        in_specs=[pl.BlockSpec(memory_space=pltpu.MemorySpace.HBM),
                  pl.BlockSpec((n, nhid), lambda p, i: (0, 0)),
                  pl.BlockSpec((1, nhid), lambda p, i: (0, 0)),
                  pl.BlockSpec((nhid, ncls), lambda p, i: (0, 0)),
                  pl.BlockSpec((1, ncls), lambda p, i: (0, 0))],
        out_specs=pl.BlockSpec((bm, ncls), lambda p, i: (i, 0)),
        out_shape=jax.ShapeDtypeStruct((n, ncls), jnp.float32),
        scratch_shapes=[pltpu.VMEM((n, ncls), jnp.float32),
                        pltpu.VMEM((_CACHE_BLOCKS, bm, n), jnp.bfloat16),
                        pltpu.VMEM((2, bm, n), jnp.float32),
                        pltpu.SemaphoreType.DMA((2,))],
        compiler_params=pltpu.CompilerParams(
            dimension_semantics=("arbitrary", "arbitrary"),
            vmem_limit_bytes=64 * 1024 * 1024),
    )(adj, s1, b1r, W2, b2r)


# speedup vs baseline: 1.0570x; 1.0551x over previous
import jax
import jax.numpy as jnp
from jax.experimental import pallas as pl
from jax.experimental.pallas import tpu as pltpu

_BM = 400
_CACHE_BLOCKS = 3


def _s1_kernel(x_ref, w1_ref, o_ref):
    o_ref[...] = jnp.dot(x_ref[...], w1_ref[...],
                         preferred_element_type=jnp.float32
                         ).astype(jnp.bfloat16)


def _fused_kernel(adj_hbm, s1_ref, b1_ref, w2_ref, b2_ref,
                  o_ref, t2_ref, cache_ref, buf_ref, sem_ref):
    p = pl.program_id(0)
    i = pl.program_id(1)
    nb = pl.num_programs(1)
    bm = o_ref.shape[0]
    c2 = 2 * _CACHE_BLOCKS
    s = p * nb + i
    cur_slot = jax.lax.rem(s, 2)
    nxt_slot = jax.lax.rem(s + 1, 2)

    def block_dma(b, slot):
        return pltpu.make_async_copy(
            adj_hbm.at[pl.ds(b * bm, bm), :],
            buf_ref.at[slot],
            sem_ref.at[slot])

    @pl.when(s == 0)
    def _prime():
        block_dma(0, 0).start()

    ni = jnp.where(i == nb - 1, 0, i + 1)
    nph = jnp.where(i == nb - 1, p + 1, p)
    nxt_cached = (nph == 1) & (ni % 2 == 1) & (ni < c2)
    nxt_valid = jnp.logical_not((p == 1) & (i == nb - 1))

    @pl.when(nxt_valid & jnp.logical_not(nxt_cached))
    def _prefetch():
        block_dma(ni, nxt_slot).start()

    is_cached = (i % 2 == 1) & (i < c2)

    @pl.when(p == 0)
    def _layer1():
        block_dma(i, cur_slot).wait()

        @pl.when(is_cached)
        def _stash():
            cache_ref[i // 2] = buf_ref[cur_slot].astype(jnp.bfloat16)

        acc = jnp.dot(buf_ref[cur_slot].astype(jnp.bfloat16), s1_ref[...],
                      preferred_element_type=jnp.float32)
        h = jnp.maximum(acc + b1_ref[...], 0.0)
        t2_ref[pl.ds(i * bm, bm), :] = jnp.dot(
            h, w2_ref[...], preferred_element_type=jnp.float32
        ).astype(jnp.bfloat16)

    def _layer2_epilogue(zz):
        z = zz + b2_ref[...]
        m = jnp.max(z, axis=1, keepdims=True)
        e = jnp.exp(z - m)
        lse = jnp.log(jnp.sum(e, axis=1, keepdims=True)) + m
        o_ref[...] = z - lse

    @pl.when((p == 1) & is_cached)
    def _layer2_cached():
        _layer2_epilogue(jnp.dot(cache_ref[i // 2], t2_ref[...],
                                 preferred_element_type=jnp.float32))

    @pl.when((p == 1) & jnp.logical_not(is_cached))
    def _layer2_streamed():
        block_dma(i, cur_slot).wait()
        _layer2_epilogue(jnp.dot(buf_ref[cur_slot].astype(jnp.bfloat16),
                                 t2_ref[...],
                                 preferred_element_type=jnp.float32))


def kernel(x, adj, W1, b1, W2, b2):
    n, nfeat = x.shape
    nhid = W1.shape[1]
    ncls = W2.shape[1]
    bm = _BM
    b1r = b1.reshape(1, nhid)
    b2r = b2.reshape(1, ncls)
    BS1 = 2000

    s1 = pl.pallas_call(
        _s1_kernel,
        grid=(n // BS1,),
        in_specs=[pl.BlockSpec((BS1, nfeat), lambda i: (i, 0)),
                  pl.BlockSpec((nfeat, nhid), lambda i: (0, 0))],
        out_specs=pl.BlockSpec((BS1, nhid), lambda i: (i, 0)),
        out_shape=jax.ShapeDtypeStruct((n, nhid), jnp.bfloat16),
    )(x, W1)

    return pl.pallas_call(
        _fused_kernel,
        grid=(2, n // bm),
        in_specs=[pl.BlockSpec(memory_space=pltpu.MemorySpace.HBM),
                  pl.BlockSpec((n, nhid), lambda p, i: (0, 0)),
                  pl.BlockSpec((1, nhid), lambda p, i: (0, 0)),
                  pl.BlockSpec((nhid, ncls), lambda p, i: (0, 0)),
                  pl.BlockSpec((1, ncls), lambda p, i: (0, 0))],
        out_specs=pl.BlockSpec((bm, ncls), lambda p, i: (i, 0)),
        out_shape=jax.ShapeDtypeStruct((n, ncls), jnp.float32),
        scratch_shapes=[pltpu.VMEM((n, ncls), jnp.bfloat16),
                        pltpu.VMEM((_CACHE_BLOCKS, bm, n), jnp.bfloat16),
                        pltpu.VMEM((2, bm, n), jnp.float32),
                        pltpu.SemaphoreType.DMA((2,))],
        compiler_params=pltpu.CompilerParams(
            dimension_semantics=("arbitrary", "arbitrary"),
            vmem_limit_bytes=64 * 1024 * 1024),
    )(adj, s1, b1r, W2, b2r)


# speedup vs baseline: 1.0765x; 1.0184x over previous
import jax
import jax.numpy as jnp
from jax.experimental import pallas as pl
from jax.experimental.pallas import tpu as pltpu

_BM = 400
_CACHE_BLOCKS = 3


def _s1_kernel(x_ref, w1_ref, o_ref):
    o_ref[...] = jnp.dot(x_ref[...], w1_ref[...],
                         preferred_element_type=jnp.float32
                         ).astype(jnp.bfloat16)


def _make_fused_kernel(nb):
    cb = _CACHE_BLOCKS
    ns = nb + (nb - cb) + 1
    ncache0 = nb - cb

    def _fused_kernel(adj_hbm, s1_ref, b1_ref, w2_ref, b2_ref,
                      o_hbm, t2_ref, cache_ref, buf_ref, stage_ref, sem_ref):
        s = pl.program_id(0)
        bm = _BM
        cur_slot = jax.lax.rem(s, 2)
        nxt_slot = jax.lax.rem(s + 1, 2)

        def need_b(t):
            return jnp.where(t < nb, t, t - nb)

        def block_dma(b, slot):
            return pltpu.make_async_copy(
                adj_hbm.at[pl.ds(b * bm, bm), :],
                buf_ref.at[slot],
                sem_ref.at[slot])

        @pl.when(s == 0)
        def _prime():
            block_dma(0, 0).start()

        @pl.when(s < ns - 2)
        def _prefetch():
            block_dma(need_b(s + 1), nxt_slot).start()

        @pl.when(s < nb)
        def _layer1():
            block_dma(need_b(s), cur_slot).wait()

            @pl.when(s >= ncache0)
            def _stash():
                cache_ref[pl.ds((s - ncache0) * bm, bm), :] = (
                    buf_ref[cur_slot].astype(jnp.bfloat16))

            acc = jnp.dot(buf_ref[cur_slot].astype(jnp.bfloat16), s1_ref[...],
                          preferred_element_type=jnp.float32)
            h = jnp.maximum(acc + b1_ref[...], 0.0)
            t2_ref[pl.ds(s * bm, bm), :] = jnp.dot(
                h, w2_ref[...], preferred_element_type=jnp.float32
            ).astype(jnp.bfloat16)

        def _softmax_rows(zz):
            z = zz + b2_ref[...]
            m = jnp.max(z, axis=1, keepdims=True)
            e = jnp.exp(z - m)
            lse = jnp.log(jnp.sum(e, axis=1, keepdims=True)) + m
            return z - lse

        @pl.when((s >= nb) & (s < ns - 1))
        def _layer2_streamed():
            block_dma(need_b(s), cur_slot).wait()
            zz = jnp.dot(buf_ref[cur_slot].astype(jnp.bfloat16), t2_ref[...],
                         preferred_element_type=jnp.float32)
            stage_ref[pl.ds(0, bm), :] = _softmax_rows(zz)
            b = s - nb
            pltpu.sync_copy(stage_ref.at[pl.ds(0, bm), :],
                            o_hbm.at[pl.ds(b * bm, bm), :])

        @pl.when(s == ns - 1)
        def _layer2_cached():
            zz = jnp.dot(cache_ref[...], t2_ref[...],
                         preferred_element_type=jnp.float32)
            stage_ref[...] = _softmax_rows(zz)
            pltpu.sync_copy(stage_ref,
                            o_hbm.at[pl.ds(ncache0 * bm, cb * bm), :])

    return _fused_kernel, ns


def kernel(x, adj, W1, b1, W2, b2):
    n, nfeat = x.shape
    nhid = W1.shape[1]
    ncls = W2.shape[1]
    bm = _BM
    cb = _CACHE_BLOCKS
    nb = n // bm
    b1r = b1.reshape(1, nhid)
    b2r = b2.reshape(1, ncls)
    BS1 = 2000

    s1 = pl.pallas_call(
        _s1_kernel,
        grid=(n // BS1,),
        in_specs=[pl.BlockSpec((BS1, nfeat), lambda i: (i, 0)),
                  pl.BlockSpec((nfeat, nhid), lambda i: (0, 0))],
        out_specs=pl.BlockSpec((BS1, nhid), lambda i: (i, 0)),
        out_shape=jax.ShapeDtypeStruct((n, nhid), jnp.bfloat16),
    )(x, W1)

    fused, ns = _make_fused_kernel(nb)

    return pl.pallas_call(
        fused,
        grid=(ns,),
        in_specs=[pl.BlockSpec(memory_space=pltpu.MemorySpace.HBM),
                  pl.BlockSpec((n, nhid), lambda s: (0, 0)),
                  pl.BlockSpec((1, nhid), lambda s: (0, 0)),
                  pl.BlockSpec((nhid, ncls), lambda s: (0, 0)),
                  pl.BlockSpec((1, ncls), lambda s: (0, 0))],
        out_specs=pl.BlockSpec(memory_space=pltpu.MemorySpace.HBM),
        out_shape=jax.ShapeDtypeStruct((n, ncls), jnp.float32),
        scratch_shapes=[pltpu.VMEM((n, ncls), jnp.bfloat16),
                        pltpu.VMEM((cb * bm, n), jnp.bfloat16),
                        pltpu.VMEM((2, bm, n), jnp.float32),
                        pltpu.VMEM((cb * bm, ncls), jnp.float32),
                        pltpu.SemaphoreType.DMA((2,))],
        compiler_params=pltpu.CompilerParams(
            dimension_semantics=("arbitrary",),
            vmem_limit_bytes=64 * 1024 * 1024),
    )(adj, s1, b1r, W2, b2r)


# speedup vs baseline: 1.0862x; 1.0090x over previous
import jax
import jax.numpy as jnp
from jax.experimental import pallas as pl
from jax.experimental.pallas import tpu as pltpu

_BM = 400
_CACHE_BLOCKS = 3


def _s1_kernel(x_ref, w1_ref, o_ref):
    o_ref[...] = jnp.dot(x_ref[...], w1_ref[...],
                         preferred_element_type=jnp.float32
                         ).astype(jnp.bfloat16)


def _make_fused_kernel(nb):
    cb = _CACHE_BLOCKS
    ns = nb + (nb - cb) + 1
    ncache0 = nb - cb

    def _fused_kernel(adj_hbm, s1_ref, b1_ref, w2_ref, b2_ref,
                      o_hbm, t2_ref, cache_ref, buf_ref, stage_ref, sem_ref):
        s = pl.program_id(0)
        bm = _BM
        cur_slot = jnp.where(s <= nb, s % 2, (s - 1) % 2)

        def block_dma(b, slot):
            return pltpu.make_async_copy(
                adj_hbm.at[pl.ds(b * bm, bm), :],
                buf_ref.at[slot],
                sem_ref.at[slot])

        @pl.when(s == 0)
        def _prime():
            block_dma(0, 0).start()

        @pl.when(s < nb - 1)
        def _prefetch_p0():
            block_dma(s + 1, (s + 1) % 2).start()

        @pl.when(s == nb - 1)
        def _prefetch_j0():
            block_dma(0, nb % 2).start()

        if nb - cb >= 2:
            @pl.when(s == nb)
            def _prefetch_j1():
                block_dma(1, (nb + 1) % 2).start()

        @pl.when((s >= nb + 2) & (s - nb <= nb - cb - 1))
        def _prefetch_stream():
            block_dma(s - nb, s % 2).start()

        @pl.when(s < nb)
        def _layer1():
            block_dma(s, cur_slot).wait()

            @pl.when(s >= ncache0)
            def _stash():
                cache_ref[pl.ds((s - ncache0) * bm, bm), :] = (
                    buf_ref[cur_slot].astype(jnp.bfloat16))

            acc = jnp.dot(buf_ref[cur_slot].astype(jnp.bfloat16), s1_ref[...],
                          preferred_element_type=jnp.float32)
            h = jnp.maximum(acc + b1_ref[...], 0.0)
            t2_ref[pl.ds(s * bm, bm), :] = jnp.dot(
                h, w2_ref[...], preferred_element_type=jnp.float32
            ).astype(jnp.bfloat16)

        def _softmax_rows(zz):
            z = zz + b2_ref[...]
            m = jnp.max(z, axis=1, keepdims=True)
            e = jnp.exp(z - m)
            lse = jnp.log(jnp.sum(e, axis=1, keepdims=True)) + m
            return z - lse

        @pl.when(s > nb)
        def _layer2_streamed():
            b = s - nb - 1
            block_dma(b, cur_slot).wait()
            zz = jnp.dot(buf_ref[cur_slot].astype(jnp.bfloat16), t2_ref[...],
                         preferred_element_type=jnp.float32)
            stage_ref[pl.ds(0, bm), :] = _softmax_rows(zz)
            pltpu.sync_copy(stage_ref.at[pl.ds(0, bm), :],
                            o_hbm.at[pl.ds(b * bm, bm), :])

        @pl.when(s == nb)
        def _layer2_cached():
            zz = jnp.dot(cache_ref[...], t2_ref[...],
                         preferred_element_type=jnp.float32)
            stage_ref[...] = _softmax_rows(zz)
            pltpu.sync_copy(stage_ref,
                            o_hbm.at[pl.ds(ncache0 * bm, cb * bm), :])

    return _fused_kernel, ns


def kernel(x, adj, W1, b1, W2, b2):
    n, nfeat = x.shape
    nhid = W1.shape[1]
    ncls = W2.shape[1]
    bm = _BM
    cb = _CACHE_BLOCKS
    nb = n // bm
    b1r = b1.reshape(1, nhid)
    b2r = b2.reshape(1, ncls)
    BS1 = 2000

    s1 = pl.pallas_call(
        _s1_kernel,
        grid=(n // BS1,),
        in_specs=[pl.BlockSpec((BS1, nfeat), lambda i: (i, 0)),
                  pl.BlockSpec((nfeat, nhid), lambda i: (0, 0))],
        out_specs=pl.BlockSpec((BS1, nhid), lambda i: (i, 0)),
        out_shape=jax.ShapeDtypeStruct((n, nhid), jnp.bfloat16),
    )(x, W1)

    fused, ns = _make_fused_kernel(nb)

    return pl.pallas_call(
        fused,
        grid=(ns,),
        in_specs=[pl.BlockSpec(memory_space=pltpu.MemorySpace.HBM),
                  pl.BlockSpec((n, nhid), lambda s: (0, 0)),
                  pl.BlockSpec((1, nhid), lambda s: (0, 0)),
                  pl.BlockSpec((nhid, ncls), lambda s: (0, 0)),
                  pl.BlockSpec((1, ncls), lambda s: (0, 0))],
        out_specs=pl.BlockSpec(memory_space=pltpu.MemorySpace.HBM),
        out_shape=jax.ShapeDtypeStruct((n, ncls), jnp.float32),
        scratch_shapes=[pltpu.VMEM((n, ncls), jnp.bfloat16),
                        pltpu.VMEM((cb * bm, n), jnp.bfloat16),
                        pltpu.VMEM((2, bm, n), jnp.float32),
                        pltpu.VMEM((cb * bm, ncls), jnp.float32),
                        pltpu.SemaphoreType.DMA((2,))],
        compiler_params=pltpu.CompilerParams(
            dimension_semantics=("arbitrary",),
            vmem_limit_bytes=64 * 1024 * 1024),
    )(adj, s1, b1r, W2, b2r)
